# Initial kernel scaffold; baseline (speedup 1.0000x reference)
import jax
import jax.numpy as jnp
from jax.experimental import pallas as pl


def kernel(x, edge_index, W1, b1, W2, b2, Wd, bd):
    raise NotImplementedError("write your pallas kernel here")


# speedup vs baseline: 4.8366x; 4.8366x over previous
import functools
import jax
import jax.numpy as jnp
from jax import lax
from jax.experimental import pallas as pl
from jax.experimental.pallas import tpu as pltpu
from jax.experimental.pallas import tpu_sc as plsc

_N = 10000
_F = 128
_H = 128
_E = 320000
_NC = 2
_NS = 16
_NPAD = 10240
_NH = _NPAD // _NC
_NHJ = _NH + 128
_NHJD = 6144
_ZPS = _NHJD // _NS
_RPS = _NH // _NS
_CB = 128
_EW = _E // _NS
_CH = 160
_EWP = _CH * _CB
_BM = 1024
_GRID = _NPAD // _BM

_mesh = plsc.VectorSubcoreMesh(
    core_axis_name="c", subcore_axis_name="s", num_cores=_NC, num_subcores=_NS
)


def _remap_dst(dst_v, cid):
    base = cid * _NH

    @pl.loop(0, _CH)
    def _rows(i):
        for j in range(_CB // 16):
            v = dst_v[i, pl.ds(j * 16, 16)] - base
            ok = jnp.logical_and(v >= 0, v < _NH)
            dst_v[i, pl.ds(j * 16, 16)] = jnp.where(ok, v, _NH)


@functools.partial(
    pl.kernel,
    out_type=jax.ShapeDtypeStruct((_NC, _NHJD), jnp.float32),
    mesh=_mesh,
    scratch_types=[
        pltpu.VMEM((_CH, _CB), jnp.int32),
        pltpu.VMEM((_CB,), jnp.float32),
        pltpu.VMEM((_ZPS,), jnp.float32),
        pltpu.VMEM_SHARED((_NHJD,), jnp.float32),
    ],
)
def _deg_kernel(dst_hbm, out_hbm, idx_v, ones_v, zeros_v, acc_sh):
    cid = lax.axis_index("c")
    sid = lax.axis_index("s")
    one16 = jnp.ones((16,), jnp.float32)
    zero16 = jnp.zeros((16,), jnp.float32)
    for i in range(_CB // 16):
        ones_v[pl.ds(i * 16, 16)] = one16

    @pl.loop(0, _ZPS // 16)
    def _zfill(i):
        zeros_v[pl.ds(i * 16, 16)] = zero16

    pltpu.sync_copy(zeros_v, acc_sh.at[pl.ds(sid * _ZPS, _ZPS)])
    plsc.subcore_barrier()
    pltpu.sync_copy(dst_hbm.at[sid], idx_v)
    _remap_dst(idx_v, cid)

    @pl.loop(0, _CH)
    def _scat(k):
        pltpu.sync_copy(ones_v, acc_sh.at[idx_v.at[k]], add=True)

    plsc.subcore_barrier()

    @pl.when(sid == 0)
    def _out():
        pltpu.sync_copy(acc_sh, out_hbm.at[cid])


@functools.partial(
    pl.kernel,
    out_type=jax.ShapeDtypeStruct((_NC, _NS, _RPS, _H), jnp.float32),
    mesh=_mesh,
    scratch_types=[
        pltpu.VMEM((_CH, _CB), jnp.int32),
        pltpu.VMEM((_CH, _CB), jnp.int32),
        pltpu.VMEM((_CB, _H), jnp.float32),
        pltpu.VMEM((_CB // 2, _H), jnp.float32),
        pltpu.VMEM_SHARED((_NHJ, _H), jnp.float32),
        pltpu.SemaphoreType.DMA,
    ],
)
def _agg_kernel(y_hbm, src_hbm, dst_hbm, out_hbm,
                src_v, dst_v, rows_v, zeros_v, acc_sh, sem):
    cid = lax.axis_index("c")
    sid = lax.axis_index("s")
    zero16 = jnp.zeros((16,), jnp.float32)

    @pl.loop(0, _CB // 2)
    def _zfill(i):
        for j in range(_H // 16):
            zeros_v[i, pl.ds(j * 16, 16)] = zero16

    for j in range(_RPS // (_CB // 2)):
        pltpu.sync_copy(
            zeros_v, acc_sh.at[pl.ds(sid * _RPS + j * (_CB // 2), _CB // 2)]
        )
    plsc.subcore_barrier()

    pltpu.sync_copy(src_hbm.at[sid], src_v)
    pltpu.sync_copy(dst_hbm.at[sid], dst_v)
    _remap_dst(dst_v, cid)

    @pl.loop(0, _CH)
    def _scat(k):
        pltpu.async_copy(y_hbm.at[src_v.at[k]], rows_v, sem).wait()
        pltpu.sync_copy(rows_v, acc_sh.at[dst_v.at[k]], add=True)

    plsc.subcore_barrier()
    pltpu.sync_copy(acc_sh.at[pl.ds(sid * _RPS, _RPS)], out_hbm.at[cid].at[sid])


def _tc1_body(x_ref, deg_ref, w_ref, y_ref):
    dinv = lax.rsqrt(deg_ref[...] + 1.0)
    y_ref[...] = jnp.dot(
        x_ref[...], w_ref[...], preferred_element_type=jnp.float32
    ) * dinv


def _tc2_body(p_ref, y_ref, deg_ref, b_ref, w_ref, y2_ref):
    dinv = lax.rsqrt(deg_ref[...] + 1.0)
    h = jnp.maximum((p_ref[...] + y_ref[...]) * dinv + b_ref[...], 0.0)
    y2_ref[...] = jnp.dot(
        h, w_ref[...], preferred_element_type=jnp.float32
    ) * dinv


def _tc3_body(p_ref, y_ref, deg_ref, b_ref, wd_ref, out_ref):
    dinv = lax.rsqrt(deg_ref[...] + 1.0)
    h = jnp.maximum((p_ref[...] + y_ref[...]) * dinv + b_ref[...], 0.0)
    out_ref[...] = jnp.sum(h * wd_ref[...], axis=1).reshape(_BM // 128, 128)


_blk_spec = pl.BlockSpec((_BM, _H), lambda i: (i, 0))
_deg_spec = pl.BlockSpec((_BM, 1), lambda i: (i, 0))
_row_spec = pl.BlockSpec((1, _H), lambda i: (0, 0))
_mat_spec = pl.BlockSpec((_H, _H), lambda i: (0, 0))
_blk_shape = jax.ShapeDtypeStruct((_NPAD, _H), jnp.float32)

_tc1 = pl.pallas_call(
    _tc1_body,
    grid=(_GRID,),
    in_specs=[_blk_spec, _deg_spec, _mat_spec],
    out_specs=_blk_spec,
    out_shape=_blk_shape,
)

_tc2 = pl.pallas_call(
    _tc2_body,
    grid=(_GRID,),
    in_specs=[_blk_spec, _blk_spec, _deg_spec, _row_spec, _mat_spec],
    out_specs=_blk_spec,
    out_shape=_blk_shape,
)

_tc3 = pl.pallas_call(
    _tc3_body,
    grid=(_GRID,),
    in_specs=[_blk_spec, _blk_spec, _deg_spec, _row_spec, _row_spec],
    out_specs=pl.BlockSpec((_BM // 128, 128), lambda i: (i, 0)),
    out_shape=jax.ShapeDtypeStruct((_NPAD // 128, 128), jnp.float32),
)


@jax.jit
def kernel(x, edge_index, W1, b1, W2, b2, Wd, bd):
    src = edge_index[0].astype(jnp.int32)
    dst = edge_index[1].astype(jnp.int32)
    srcw = jnp.pad(src.reshape(_NS, _EW), ((0, 0), (0, _EWP - _EW)))
    srcw = srcw.reshape(_NS, _CH, _CB)
    dstw = jnp.pad(
        dst.reshape(_NS, _EW), ((0, 0), (0, _EWP - _EW)), constant_values=_N
    ).reshape(_NS, _CH, _CB)

    deg = _deg_kernel(dstw)[:, :_NH].reshape(_NPAD, 1)
    xp = jnp.pad(x, ((0, _NPAD - _N), (0, 0)))
    y1 = _tc1(xp, deg, W1)
    p1 = _agg_kernel(y1, srcw, dstw).reshape(_NPAD, _H)
    y2 = _tc2(p1, y1, deg, b1.reshape(1, _H), W2)
    p2 = _agg_kernel(y2, srcw, dstw).reshape(_NPAD, _H)
    outg = _tc3(p2, y2, deg, b2.reshape(1, _H), Wd.reshape(1, _H))
    return outg.reshape(_NPAD)[:_N] + bd[0]


# speedup vs baseline: 9.7624x; 2.0184x over previous
import functools
import jax
import jax.numpy as jnp
from jax import lax
from jax.experimental import pallas as pl
from jax.experimental.pallas import tpu as pltpu
from jax.experimental.pallas import tpu_sc as plsc

_N = 10000
_F = 128
_H = 128
_E = 320000
_NC = 2
_NS = 16
_NW = _NC * _NS
_NPAD = 10240
_RPS = _NPAD // _NS
_CB = 128
_EW = _E // _NW
_CH = 80
_EWP = _CH * _CB
_BM = 1024
_GRID = _NPAD // _BM

_mesh = plsc.VectorSubcoreMesh(
    core_axis_name="c", subcore_axis_name="s", num_cores=_NC, num_subcores=_NS
)


@functools.partial(
    pl.kernel,
    out_type=jax.ShapeDtypeStruct((_NC, _NPAD), jnp.float32),
    mesh=_mesh,
    scratch_types=[
        pltpu.VMEM((_CH, _CB), jnp.int32),
        pltpu.VMEM((_CB,), jnp.float32),
        pltpu.VMEM((_RPS,), jnp.float32),
        pltpu.VMEM_SHARED((_NPAD,), jnp.float32),
    ],
)
def _deg_kernel(dst_hbm, out_hbm, idx_v, ones_v, zeros_v, acc_sh):
    cid = lax.axis_index("c")
    sid = lax.axis_index("s")
    wid = sid * _NC + cid
    one16 = jnp.ones((16,), jnp.float32)
    zero16 = jnp.zeros((16,), jnp.float32)
    for i in range(_CB // 16):
        ones_v[pl.ds(i * 16, 16)] = one16

    @pl.loop(0, _RPS // 16)
    def _zfill(i):
        zeros_v[pl.ds(i * 16, 16)] = zero16

    pltpu.sync_copy(zeros_v, acc_sh.at[pl.ds(sid * _RPS, _RPS)])
    plsc.subcore_barrier()
    pltpu.sync_copy(dst_hbm.at[wid], idx_v)

    @pl.loop(0, _CH)
    def _scat(k):
        pltpu.sync_copy(ones_v, acc_sh.at[idx_v.at[k]], add=True)

    plsc.subcore_barrier()

    @pl.when(sid == 0)
    def _out():
        pltpu.sync_copy(acc_sh, out_hbm.at[cid])


@functools.partial(
    pl.kernel,
    out_type=jax.ShapeDtypeStruct((_NC, _NS, _RPS, _H), jnp.float32),
    mesh=_mesh,
    scratch_types=[
        pltpu.VMEM((_CH, _CB), jnp.int32),
        pltpu.VMEM((_CH, _CB), jnp.int32),
        pltpu.VMEM((_CB, _H), jnp.float32),
        pltpu.VMEM_SHARED((_NPAD, _H), jnp.float32),
        pltpu.SemaphoreType.DMA,
    ],
)
def _agg_kernel(y_hbm, src_hbm, dst_hbm, out_hbm,
                src_v, dst_v, rows_v, acc_sh, sem):
    cid = lax.axis_index("c")
    sid = lax.axis_index("s")
    wid = sid * _NC + cid
    zero16 = jnp.zeros((16,), jnp.float32)

    @pl.loop(0, _CB)
    def _zfill(i):
        for j in range(_H // 16):
            rows_v[i, pl.ds(j * 16, 16)] = zero16

    for j in range(_RPS // _CB):
        pltpu.sync_copy(rows_v, acc_sh.at[pl.ds(sid * _RPS + j * _CB, _CB)])
    plsc.subcore_barrier()

    pltpu.sync_copy(src_hbm.at[wid], src_v)
    pltpu.sync_copy(dst_hbm.at[wid], dst_v)

    @pl.loop(0, _CH)
    def _scat(k):
        pltpu.async_copy(y_hbm.at[src_v.at[k]], rows_v, sem).wait()
        pltpu.sync_copy(rows_v, acc_sh.at[dst_v.at[k]], add=True)

    plsc.subcore_barrier()
    pltpu.sync_copy(acc_sh.at[pl.ds(sid * _RPS, _RPS)], out_hbm.at[cid].at[sid])


def _dinv_of(d0_ref, d1_ref):
    return lax.rsqrt(d0_ref[...] + d1_ref[...] + 1.0)


def _tc1_body(x_ref, d0_ref, d1_ref, w_ref, y_ref):
    dinv = _dinv_of(d0_ref, d1_ref)
    y_ref[...] = jnp.dot(
        x_ref[...], w_ref[...], preferred_element_type=jnp.float32
    ) * dinv


def _tc2_body(pa_ref, pb_ref, y_ref, d0_ref, d1_ref, b_ref, w_ref, y2_ref):
    dinv = _dinv_of(d0_ref, d1_ref)
    h = jnp.maximum(
        (pa_ref[...] + pb_ref[...] + y_ref[...]) * dinv + b_ref[...], 0.0
    )
    y2_ref[...] = jnp.dot(
        h, w_ref[...], preferred_element_type=jnp.float32
    ) * dinv


def _tc3_body(pa_ref, pb_ref, y_ref, d0_ref, d1_ref, b_ref, wd_ref, out_ref):
    dinv = _dinv_of(d0_ref, d1_ref)
    h = jnp.maximum(
        (pa_ref[...] + pb_ref[...] + y_ref[...]) * dinv + b_ref[...], 0.0
    )
    out_ref[...] = jnp.sum(h * wd_ref[...], axis=1).reshape(_BM // 128, 128)


_blk_spec = pl.BlockSpec((_BM, _H), lambda i: (i, 0))
_deg_spec = pl.BlockSpec((_BM, 1), lambda i: (i, 0))
_row_spec = pl.BlockSpec((1, _H), lambda i: (0, 0))
_mat_spec = pl.BlockSpec((_H, _H), lambda i: (0, 0))
_blk_shape = jax.ShapeDtypeStruct((_NPAD, _H), jnp.float32)

_tc1 = pl.pallas_call(
    _tc1_body,
    grid=(_GRID,),
    in_specs=[_blk_spec, _deg_spec, _deg_spec, _mat_spec],
    out_specs=_blk_spec,
    out_shape=_blk_shape,
)

_tc2 = pl.pallas_call(
    _tc2_body,
    grid=(_GRID,),
    in_specs=[_blk_spec, _blk_spec, _blk_spec, _deg_spec, _deg_spec,
              _row_spec, _mat_spec],
    out_specs=_blk_spec,
    out_shape=_blk_shape,
)

_tc3 = pl.pallas_call(
    _tc3_body,
    grid=(_GRID,),
    in_specs=[_blk_spec, _blk_spec, _blk_spec, _deg_spec, _deg_spec,
              _row_spec, _row_spec],
    out_specs=pl.BlockSpec((_BM // 128, 128), lambda i: (i, 0)),
    out_shape=jax.ShapeDtypeStruct((_NPAD // 128, 128), jnp.float32),
)


@jax.jit
def kernel(x, edge_index, W1, b1, W2, b2, Wd, bd):
    src = edge_index[0].astype(jnp.int32)
    dst = edge_index[1].astype(jnp.int32)
    srcw = jnp.pad(src.reshape(_NW, _EW), ((0, 0), (0, _EWP - _EW)))
    srcw = srcw.reshape(_NW, _CH, _CB)
    dstw = jnp.pad(
        dst.reshape(_NW, _EW), ((0, 0), (0, _EWP - _EW)), constant_values=_N
    ).reshape(_NW, _CH, _CB)

    degp = _deg_kernel(dstw)
    d0 = degp[0].reshape(_NPAD, 1)
    d1 = degp[1].reshape(_NPAD, 1)
    xp = jnp.pad(x, ((0, _NPAD - _N), (0, 0)))
    y1 = _tc1(xp, d0, d1, W1)
    p1 = _agg_kernel(y1, srcw, dstw)
    p1a = p1[0].reshape(_NPAD, _H)
    p1b = p1[1].reshape(_NPAD, _H)
    y2 = _tc2(p1a, p1b, y1, d0, d1, b1.reshape(1, _H), W2)
    p2 = _agg_kernel(y2, srcw, dstw)
    p2a = p2[0].reshape(_NPAD, _H)
    p2b = p2[1].reshape(_NPAD, _H)
    outg = _tc3(p2a, p2b, y2, d0, d1, b2.reshape(1, _H), Wd.reshape(1, _H))
    return outg.reshape(_NPAD)[:_N] + bd[0]
